# baseline (speedup 1.0000x reference)
import functools
import jax
import jax.numpy as jnp
from jax import lax
from jax.experimental import pallas as pl
from jax.experimental.pallas import tpu as pltpu
from jax.experimental.pallas import tpu_sc as plsc

_N = 4096
_OUT = 8192
_CHUNK = 256
_E1LEN = _N + 16
_PAD = _N
_E2LEN = 8224


def _sc_body(d_hbm, o_hbm, d_v, e1_v, e2p_v, o_v):
    wid = lax.axis_index("s") * 2 + lax.axis_index("c")
    base = wid * _CHUNK

    pltpu.sync_copy(d_hbm, d_v)

    zv = jnp.zeros((16,), jnp.float32)
    e1_v[pl.ds(_N, 16)] = zv

    def zero_body(i, c):
        e2p_v[pl.ds(i * 16, 16)] = zv
        return c
    lax.fori_loop(0, _PAD // 16, zero_body, 0)
    e2p_v[pl.ds(_PAD + _N, 16)] = zv
    e2p_v[pl.ds(_PAD + _N + 16, 16)] = zv

    def exp_body(i, zz):
        z1, z2 = zz
        c1 = jnp.exp(d_v[0, pl.ds(i * 16, 16)])
        e1_v[pl.ds(i * 16, 16)] = c1
        c2 = jnp.exp(d_v[1, pl.ds(i * 16, 16)])
        e2p_v[pl.ds(_PAD + i * 16, 16)] = c2
        return (z1 + c1, z2 + c2)

    z1, z2 = lax.fori_loop(0, _N // 16, exp_body, (zv, zv))
    z1s = z1[0]
    z2s = z2[0]
    for lane in range(1, 16):
        z1s = z1s + z1[lane]
        z2s = z2s + z2[lane]
    scale = 1.0 / jnp.full((16,), z1s * z2s, jnp.float32)

    for v in range(_CHUNK // 16):
        tv = base + 16 * v
        n_lo = (jnp.maximum(0, tv - (_N - 1)) // 16) * 16
        n_hi = jnp.minimum(_N, tv + 16)
        trips = (n_hi - n_lo + 15) // 16

        def conv_body(k, acc, tv=tv, n_lo=n_lo):
            n0 = n_lo + k * 16
            e1vec = e1_v[pl.ds(n0, 16)]
            for u in range(16):
                w = e2p_v[pl.ds(tv - (n0 + u) + _PAD, 16)]
                acc = acc + e1vec[u] * w
            return acc

        acc = lax.fori_loop(0, trips, conv_body, zv)
        o_v[pl.ds(16 * v, 16)] = acc * scale

    pltpu.sync_copy(o_v, o_hbm.at[pl.ds(base, _CHUNK)])


def kernel(d_distr):
    mesh = plsc.VectorSubcoreMesh(core_axis_name="c", subcore_axis_name="s")
    f = functools.partial(
        pl.kernel,
        out_type=jax.ShapeDtypeStruct((_OUT,), jnp.float32),
        mesh=mesh,
        scratch_types=[
            pltpu.VMEM((2, _N), jnp.float32),
            pltpu.VMEM((_E1LEN,), jnp.float32),
            pltpu.VMEM((_E2LEN,), jnp.float32),
            pltpu.VMEM((_CHUNK,), jnp.float32),
        ],
    )(_sc_body)
    return f(d_distr)[: 2 * _N - 1]


# speedup vs baseline: 39.4762x; 39.4762x over previous
import jax
import jax.numpy as jnp
from jax.experimental import pallas as pl
from jax.experimental.pallas import tpu as pltpu

_N = 4096
_L = 64
_P = _N // _L
_W = 2 * _N
_WG = _N + 512


def _shift_right(x, k):
    pad = jnp.zeros(x.shape[:-1] + (k,), x.dtype)
    return jnp.concatenate([pad, x[..., :-k]], axis=-1)


def _conv_body(d_ref, o_ref, g_ref, a_ref):
    e = jnp.exp(d_ref[...])
    zz = jnp.sum(e[0:1, :]) * jnp.sum(e[1:2, :])
    p1 = e[0:1, :]
    p2 = e[1:2, :]

    for q in range(_P):
        a_ref[q : q + 1, :] = p1[:, _L * q : _L * (q + 1)]
    a = a_ref[...]

    g8 = jnp.broadcast_to(
        jnp.concatenate([p2, jnp.zeros((1, _WG - _N), jnp.float32)], axis=1),
        (8, _WG))
    srow = jax.lax.broadcasted_iota(jnp.int32, (8, 1), 0)
    for bit in range(3):
        k = 1 << bit
        g8 = jnp.where((srow >> bit) & 1 == 1, _shift_right(g8, k), g8)

    g_ref[0:8, :] = g8
    for blk in range(1, 8):
        off = 8 * blk
        g_ref[8 * blk : 8 * blk + 8, :off] = jnp.zeros((8, off), jnp.float32)
        g_ref[8 * blk : 8 * blk + 8, off:] = g8[:, : _WG - off]

    m = jax.lax.dot_general(a, g_ref[...], (((1,), (0,)), ((), ())),
                            preferred_element_type=jnp.float32)

    qrow = jax.lax.broadcasted_iota(jnp.int32, (_P, 1), 0)
    for bit in range(3):
        k = _L << bit
        m = jnp.where((qrow >> bit) & 1 == 1, _shift_right(m, k), m)
    ra = jax.lax.broadcasted_iota(jnp.int32, (8, _P), 0)
    ri = jax.lax.broadcasted_iota(jnp.int32, (8, _P), 1)
    red = (ra == (ri >> 3)).astype(jnp.float32)
    m8 = jax.lax.dot_general(red, m, (((1,), (0,)), ((), ())),
                             preferred_element_type=jnp.float32)
    m8 = jnp.concatenate([m8, jnp.zeros((8, _W - _WG), jnp.float32)], axis=1)
    arow = jax.lax.broadcasted_iota(jnp.int32, (8, 1), 0)
    for bit in range(3):
        k = 512 << bit
        m8 = jnp.where((arow >> bit) & 1 == 1, _shift_right(m8, k), m8)

    o_ref[...] = (jnp.sum(m8, axis=0) * (1.0 / zz))[: 2 * _N - 1]


def kernel(d_distr):
    return pl.pallas_call(
        _conv_body,
        out_shape=jax.ShapeDtypeStruct((2 * _N - 1,), jnp.float32),
        scratch_shapes=[pltpu.VMEM((_L, _WG), jnp.float32),
                        pltpu.VMEM((_P, _L), jnp.float32)],
    )(d_distr)
